# Initial kernel scaffold; baseline (speedup 1.0000x reference)
import jax
import jax.numpy as jnp
from jax.experimental import pallas as pl


def kernel(features, edge_index, W_w, b_w, W_x, b_x):
    raise NotImplementedError("write your pallas kernel here")


# speedup vs baseline: 3.6080x; 3.6080x over previous
import functools

import jax
import jax.numpy as jnp
from jax import lax
from jax.experimental import pallas as pl
from jax.experimental.pallas import tpu as pltpu
from jax.experimental.pallas import tpu_sc as plsc

N_NODES = 10000
N_EDGES = 320000
D_FEAT = 128
NCLS = 16
DOUT = 2 * NCLS

NC = 2
NS = 16
NW = NC * NS
L = 16

PER_W = N_EDGES // NW
G = 80
CSUB = 5
CH = G * CSUB
NCHUNK = PER_W // CH
NROW_IDX = N_EDGES // G


def _proj_body(f_ref, wt_ref, o_ref):
    o_ref[...] = jnp.dot(f_ref[...], wt_ref[...],
                         preferred_element_type=jnp.float32)


_proj = pl.pallas_call(
    _proj_body,
    out_shape=jax.ShapeDtypeStruct((N_NODES, DOUT), jnp.float32),
)


def _edge_body(p_hbm, src_hbm, dst_hbm, bias_hbm, w_hbm, x_hbm,
               idx_s, idx_d, rows_s, rows_d, out_w, out_x, bias_v, sem):
    wid = lax.axis_index("s") * NC + lax.axis_index("c")
    pltpu.sync_copy(bias_hbm, bias_v)
    b_lo = bias_v[pl.ds(0, L)]
    b_hi = bias_v[pl.ds(L, L)]

    def chunk_body(c, carry):
        ebase = pl.multiple_of(wid * PER_W + c * CH, 8)
        pltpu.sync_copy(src_hbm.at[pl.ds(ebase, CH)], idx_s)
        pltpu.sync_copy(dst_hbm.at[pl.ds(ebase, CH)], idx_d)
        cps = []
        for j in range(CSUB):
            cps.append(pltpu.async_copy(
                p_hbm.at[idx_s.at[pl.ds(j * G, G)]],
                rows_s.at[pl.ds(j * G, G)], sem))
            cps.append(pltpu.async_copy(
                p_hbm.at[idx_d.at[pl.ds(j * G, G)]],
                rows_d.at[pl.ds(j * G, G)], sem))
        for cp in cps:
            cp.wait()

        def row_body(i, acc):
            out_w[i, :] = rows_s[i, pl.ds(0, L)] - rows_d[i, pl.ds(0, L)] + b_lo
            out_x[i, :] = rows_s[i, pl.ds(L, L)] - rows_d[i, pl.ds(L, L)] + b_hi
            return acc

        lax.fori_loop(0, CH, row_body, 0)
        pltpu.sync_copy(out_w, w_hbm.at[pl.ds(ebase, CH)])
        pltpu.sync_copy(out_x, x_hbm.at[pl.ds(ebase, CH)])
        return carry

    lax.fori_loop(0, NCHUNK, chunk_body, 0)


_edge = pl.kernel(
    _edge_body,
    out_type=(jax.ShapeDtypeStruct((N_EDGES, NCLS), jnp.float32),
              jax.ShapeDtypeStruct((N_EDGES, NCLS), jnp.float32)),
    mesh=plsc.VectorSubcoreMesh(core_axis_name="c", subcore_axis_name="s",
                                num_cores=NC, num_subcores=NS),
    compiler_params=pltpu.CompilerParams(use_tc_tiling_on_sc=False),
    scratch_types=[
        pltpu.VMEM((CH,), jnp.int32),
        pltpu.VMEM((CH,), jnp.int32),
        pltpu.VMEM((CH, DOUT), jnp.float32),
        pltpu.VMEM((CH, DOUT), jnp.float32),
        pltpu.VMEM((CH, NCLS), jnp.float32),
        pltpu.VMEM((CH, NCLS), jnp.float32),
        pltpu.VMEM((DOUT,), jnp.float32),
        pltpu.SemaphoreType.DMA,
    ],
)


def kernel(features, edge_index, W_w, b_w, W_x, b_x):
    wc_t = jnp.concatenate([W_w, W_x], axis=0).T
    bias = jnp.concatenate([b_w, b_x], axis=0)
    p = _proj(features, wc_t)
    ei = edge_index.astype(jnp.int32)
    src = ei[0]
    dst = ei[1]
    w, x = _edge(p, src, dst, bias)
    return w, x
